# baseline (speedup 1.0000x reference)
import functools

import jax
import jax.numpy as jnp
from jax import lax
from jax.experimental import pallas as pl
from jax.experimental.pallas import tpu as pltpu
from jax.experimental.pallas import tpu_sc as plsc

_NC = 2
_NS = 16
_NW = _NC * _NS

_V = 1000000
_D = 32
_NFULL = 7812
_VFULL = _NFULL * 128
_TAILROWS = 16
_ROWS_PER_TABLE = _NFULL * 32 + _TAILROWS
_L_ROWS = 2 * _ROWS_PER_TABLE
_NB = 245
_G = 8
_NPAIR = 16


def _make_k1():
    mesh = plsc.VectorSubcoreMesh(core_axis_name="c", subcore_axis_name="s")

    @functools.partial(
        pl.kernel,
        out_type=jax.ShapeDtypeStruct((_L_ROWS, 128), jnp.float32),
        mesh=mesh,
        scratch_types=[
            pltpu.SemaphoreType.DMA,
        ],
    )
    def k1(t1_hbm, t2_hbm, tail1_hbm, tail2_hbm, l_hbm, sem):
        wid = lax.axis_index("s") * _NC + lax.axis_index("c")
        c0 = wid * _NB

        for t_hbm, tail_hbm, row_base in (
            (t1_hbm, tail1_hbm, 0),
            (t2_hbm, tail2_hbm, _ROWS_PER_TABLE),
        ):
            def issue(i, carry):
                c = jnp.minimum(c0 + i, _NFULL - 1)
                pltpu.async_copy(
                    t_hbm.at[:, pl.ds(c * 128, 128)],
                    l_hbm.at[pl.ds(row_base + c * _D, _D)], sem)
                return carry

            lax.fori_loop(0, _NB, issue, 0)
            pltpu.make_async_copy(
                l_hbm.at[pl.ds(0, _NB * _D)],
                l_hbm.at[pl.ds(0, _NB * _D)], sem).wait()

            @pl.when(wid == 0)
            def _():
                pltpu.sync_copy(
                    tail_hbm, l_hbm.at[pl.ds(row_base + _NFULL * _D, _TAILROWS)])

    return k1


def _make_k2(n_chunks):
    mesh = plsc.VectorSubcoreMesh(core_axis_name="c", subcore_axis_name="s")
    words_per_w = n_chunks * 128

    @functools.partial(
        pl.kernel,
        out_type=jax.ShapeDtypeStruct((_NW * words_per_w,), jnp.float32),
        mesh=mesh,
        scratch_types=[
            pltpu.VMEM((n_chunks, 128), jnp.int32),
            pltpu.VMEM((words_per_w,), jnp.float32),
            pltpu.SemaphoreType.DMA,
        ],
        compiler_params=pltpu.CompilerParams(use_tc_tiling_on_sc=False),
    )
    def k2(offs_hbm, lin_hbm, out_hbm, offs_v, cat_v, sem):
        wid = lax.axis_index("s") * _NC + lax.axis_index("c")
        pltpu.sync_copy(offs_hbm.at[wid], offs_v)
        for j in range(n_chunks):
            pltpu.async_copy(
                lin_hbm.at[offs_v.at[j]], cat_v.at[pl.ds(j * 128, 128)], sem)
        pltpu.make_async_copy(
            lin_hbm.at[pl.ds(0, words_per_w)], cat_v, sem).wait()
        pltpu.sync_copy(cat_v, out_hbm.at[pl.ds(wid * words_per_w, words_per_w)])

    return k2


@jax.jit
def _concat_lookup(x, table1, table2):
    B = x.shape[0]
    t1t = table1.T
    t2t = table2.T
    tail1 = t1t[:, _VFULL:].reshape(_TAILROWS, 128)
    tail2 = t2t[:, _VFULL:].reshape(_TAILROWS, 128)

    l_buf = _make_k1()(t1t, t2t, tail1, tail2)
    lin = l_buf.reshape(-1)

    d = (jnp.arange(_D, dtype=jnp.int32) * 128)[None, :]
    d_tail = (jnp.arange(_D, dtype=jnp.int32) * 64)[None, :]

    def offsets(e, table_word_base):
        e = e.astype(jnp.int32)
        full = ((e >> 7) * 4096 + (e & 127))[:, None] + d
        tail = (_NFULL * 4096 + (e - _VFULL))[:, None] + d_tail
        return table_word_base + jnp.where((e >= _VFULL)[:, None], tail, full)

    offs = jnp.concatenate(
        [offsets(x[:, 0], 0), offsets(x[:, 1], _ROWS_PER_TABLE * 128)], axis=1)
    n_chunks = (B * 64) // (_NW * 128)
    offs = offs.reshape(_NW, n_chunks, 128)

    out = _make_k2(n_chunks)(offs, lin)
    return out.reshape(B, 64)


def kernel(x, table1, table2):
    return _concat_lookup(x, table1, table2)


# speedup vs baseline: 29.1321x; 29.1321x over previous
import functools

import jax
import jax.numpy as jnp
from jax import lax
from jax.experimental import pallas as pl
from jax.experimental.pallas import tpu as pltpu
from jax.experimental.pallas import tpu_sc as plsc

_NC = 2
_NS = 16
_NW = _NC * _NS

_V = 1000000
_D = 32
_LANES = 1664
_JMAX = 601
_VFULL = 999936
_SPT = 4 * _JMAX
_M = 2 * _SPT
_L_ROWS = 104 * _M
_SLAB_WORDS = 8 * _LANES
_STRIPE_WORDS = _JMAX * _SLAB_WORDS
_TBL_WORDS = _SPT * _SLAB_WORDS
_NSW = 76
_NSLOT = 4


def _make_k1():
    mesh = plsc.VectorSubcoreMesh(core_axis_name="c", subcore_axis_name="s")

    @functools.partial(
        pl.kernel,
        out_type=jax.ShapeDtypeStruct((_L_ROWS, 128), jnp.float32),
        mesh=mesh,
        scratch_types=[
            pltpu.VMEM((_NSLOT, 104, 128), jnp.float32),
        ] + [pltpu.SemaphoreType.DMA] * (2 * _NSLOT),
    )
    def k1(t1_hbm, t2_hbm, tail1_hbm, tail2_hbm, l_hbm, bufs, *sems):
        sem_r = sems[:_NSLOT]
        sem_w = sems[_NSLOT:]
        wid = lax.axis_index("s") * _NC + lax.axis_index("c")
        s0w = wid * _NSW

        for t_hbm, tail_hbm, slab_base in (
            (t1_hbm, tail1_hbm, 0),
            (t2_hbm, tail2_hbm, _SPT),
        ):
            def slab(g):
                return jnp.minimum(s0w + g, _SPT - 1)

            def read(g, s):
                sid = slab(g)
                r = sid // _JMAX
                j = sid - r * _JMAX
                for tc in range(12):
                    pltpu.async_copy(
                        t_hbm.at[pl.ds(8 * r, 8),
                                 pl.ds((13 * j + tc) * 128, 128)],
                        bufs.at[s, pl.ds(8 * tc, 8)], sem_r[s])
                @pl.when(j < _JMAX - 1)
                def _():
                    pltpu.async_copy(
                        t_hbm.at[pl.ds(8 * r, 8), pl.ds((13 * j + 12) * 128, 128)],
                        bufs.at[s, pl.ds(96, 8)], sem_r[s])
                @pl.when(j == _JMAX - 1)
                def _():
                    pltpu.async_copy(
                        tail_hbm.at[pl.ds(8 * r, 8)],
                        bufs.at[s, pl.ds(96, 8)], sem_r[s])

            def wait_read(s):
                pltpu.make_async_copy(
                    l_hbm.at[pl.ds(0, 104)], bufs.at[s], sem_r[s]).wait()

            def write(g, s):
                m = slab_base + slab(g)
                pltpu.async_copy(
                    bufs.at[s], l_hbm.at[pl.ds(104 * m, 104)], sem_w[s])

            def wait_write(s):
                pltpu.make_async_copy(
                    bufs.at[s], l_hbm.at[pl.ds(0, 104)], sem_w[s]).wait()

            read(0, 0)
            read(1, 1)

            def quad(p, carry):
                for s in range(_NSLOT):
                    g = _NSLOT * p + s
                    wait_read(s)
                    write(g, s)
                    s2 = (s + 2) % _NSLOT
                    @pl.when(g >= 2)
                    def _():
                        wait_write(s2)
                    read(g + 2, s2)
                return carry

            lax.fori_loop(0, _NSW // _NSLOT, quad, 0)
            wait_read(0)
            wait_read(1)
            wait_write(2)
            wait_write(3)

    return k1


def _make_k2(n_chunks):
    mesh = plsc.VectorSubcoreMesh(core_axis_name="c", subcore_axis_name="s")
    words_per_w = n_chunks * 128

    @functools.partial(
        pl.kernel,
        out_type=jax.ShapeDtypeStruct((_NW * words_per_w,), jnp.float32),
        mesh=mesh,
        scratch_types=[
            pltpu.VMEM((n_chunks, 128), jnp.int32),
            pltpu.VMEM((words_per_w,), jnp.float32),
            pltpu.SemaphoreType.DMA,
        ],
        compiler_params=pltpu.CompilerParams(use_tc_tiling_on_sc=False),
    )
    def k2(offs_hbm, lin_hbm, out_hbm, offs_v, cat_v, sem):
        wid = lax.axis_index("s") * _NC + lax.axis_index("c")
        pltpu.sync_copy(offs_hbm.at[wid], offs_v)
        for j in range(n_chunks):
            pltpu.async_copy(
                lin_hbm.at[offs_v.at[j]], cat_v.at[pl.ds(j * 128, 128)], sem)
        pltpu.make_async_copy(
            lin_hbm.at[pl.ds(0, words_per_w)], cat_v, sem).wait()
        pltpu.sync_copy(cat_v, out_hbm.at[pl.ds(wid * words_per_w, words_per_w)])

    return k2


@jax.jit
def _concat_lookup(x, table1, table2):
    B = x.shape[0]
    t1t = table1.T
    t2t = table2.T
    tail1 = jnp.pad(t1t[:, _VFULL:], ((0, 0), (0, 64)))
    tail2 = jnp.pad(t2t[:, _VFULL:], ((0, 0), (0, 64)))

    l_buf = _make_k1()(t1t, t2t, tail1, tail2)
    lin = l_buf.reshape(-1)

    dd = jnp.arange(_D, dtype=jnp.int32)
    dterm = ((dd >> 3) * _STRIPE_WORDS + (dd & 7) * 128)[None, :]

    def offsets(e, table_word_base):
        e = e.astype(jnp.int32)
        return (table_word_base + ((e >> 7) * 1024 + (e & 127)))[:, None] + dterm

    offs = jnp.concatenate(
        [offsets(x[:, 0], 0), offsets(x[:, 1], _TBL_WORDS)], axis=1)
    n_chunks = (B * 64) // (_NW * 128)
    offs = offs.reshape(_NW, n_chunks, 128)

    out = _make_k2(n_chunks)(offs, lin)
    return out.reshape(B, 64)


def kernel(x, table1, table2):
    return _concat_lookup(x, table1, table2)
